# baseline (speedup 1.0000x reference)
import math

import jax
import jax.numpy as jnp
from jax.experimental import pallas as pl

_ORI = 9
_H = 512
_W = 512
_CH = 8
_PH = _H // _CH
_PW = _W // _CH


def _hog_body(x_ref, t_ref, d_ref, pt_ref, p_ref, o_ref):
    x = x_ref[0, 0]
    T = t_ref[...]
    Dr = d_ref[...]
    PT = pt_ref[...]
    P = p_ref[...]

    zc = jnp.zeros((_H, 1), jnp.float32)

    def lf(a):
        return jnp.concatenate([zc, a[:, :-1]], axis=1)

    def rt(a):
        return jnp.concatenate([a[:, 1:], zc], axis=1)

    sv = jnp.dot(T, x, preferred_element_type=jnp.float32)
    g0 = lf(sv) - rt(sv)
    dv = jnp.dot(Dr, x, preferred_element_type=jnp.float32)
    g1 = lf(dv) + 2.0 * dv + rt(dv)

    mag = jnp.sqrt(jnp.maximum(g0 * g0 + g1 * g1, 1e-30))

    s = [None] * 10
    s[0] = g0
    for b in range(1, 9):
        ang = b * math.pi / _ORI
        s[b] = jnp.float32(math.cos(ang)) * g0 - jnp.float32(math.sin(ang)) * g1
    s[9] = -g0

    ge = [v >= 0 for v in s]
    f = [ge[b] != ge[b + 1] for b in range(_ORI)]

    bnd = g0 == 0.0

    mh = mag.astype(jnp.bfloat16).astype(jnp.float32)
    ml = mag - mh
    zero = jnp.zeros_like(mag)

    masks = [f[0] | bnd] + [f[b] for b in range(1, 8)] + [f[8] & (~bnd), bnd]

    def rowpool(m):
        uh = jnp.where(m, mh, zero)
        ul = jnp.where(m, ml, zero)
        return (jnp.dot(PT, uh, preferred_element_type=jnp.float32)
                + jnp.dot(PT, ul, preferred_element_type=jnp.float32))

    pools = [rowpool(m) for m in masks]
    R = pools[:_ORI]
    Rz = pools[_ORI]

    inv = jnp.float32(1.0 / (_CH * _CH))
    for b in range(_ORI):
        if b == 0:
            t = R[0] + R[8] + Rz
        elif b == 1:
            t = R[1] + R[0] - Rz
        else:
            t = R[b] + R[b - 1]
        o_ref[0, b] = jnp.dot(t, P, preferred_element_type=jnp.float32,
                              precision=jax.lax.Precision.HIGHEST) * inv


def kernel(x, weight):
    n = x.shape[0]
    xb = x.astype(jnp.bfloat16)
    i = jnp.arange(_H, dtype=jnp.int32)
    d = i[:, None] - i[None, :]
    one = jnp.float32(1.0)
    zero = jnp.float32(0.0)
    T = (jnp.where(jnp.abs(d) == 1, one, zero)
         + jnp.where(d == 0, jnp.float32(2.0), zero)).astype(jnp.bfloat16)
    Dr = (jnp.where(d == 1, one, zero)
          - jnp.where(d == -1, one, zero)).astype(jnp.bfloat16)
    pr = jnp.arange(_PH, dtype=jnp.int32)
    PT = jnp.where(i[None, :] // _CH == pr[:, None], one, zero)
    P = jnp.where(i[:, None] // _CH == pr[None, :], one, zero)

    pooled = pl.pallas_call(
        _hog_body,
        grid=(n,),
        in_specs=[
            pl.BlockSpec((1, 1, _H, _W), lambda i: (i, 0, 0, 0)),
            pl.BlockSpec((_H, _H), lambda i: (0, 0)),
            pl.BlockSpec((_H, _H), lambda i: (0, 0)),
            pl.BlockSpec((_PH, _H), lambda i: (0, 0)),
            pl.BlockSpec((_H, _PW), lambda i: (0, 0)),
        ],
        out_specs=pl.BlockSpec((1, _ORI, _PH, _PW), lambda i: (i, 0, 0, 0)),
        out_shape=jax.ShapeDtypeStruct((n, _ORI, _PH, _PW), jnp.float32),
    )(xb, T, Dr, PT, P)
    return pooled.reshape(n, -1)


# speedup vs baseline: 1.1300x; 1.1300x over previous
import math

import jax
import jax.numpy as jnp
from jax.experimental import pallas as pl

_ORI = 9
_H = 512
_W = 512
_CH = 8
_PH = _H // _CH
_PW = _W // _CH


def _hog_body(x_ref, t_ref, d_ref, pt_ref, p_ref, o_ref):
    x = x_ref[0, 0]
    T = t_ref[...]
    Dr = d_ref[...]
    PT = pt_ref[...]
    P = p_ref[...]

    zc = jnp.zeros((_H, 1), jnp.float32)

    def lf(a):
        return jnp.concatenate([zc, a[:, :-1]], axis=1)

    def rt(a):
        return jnp.concatenate([a[:, 1:], zc], axis=1)

    sv = jnp.dot(T, x, preferred_element_type=jnp.float32)
    g0 = lf(sv) - rt(sv)
    dv = jnp.dot(Dr, x, preferred_element_type=jnp.float32)
    g1 = lf(dv) + 2.0 * dv + rt(dv)

    mag = jnp.sqrt(jnp.maximum(g0 * g0 + g1 * g1, 1e-30))

    s = [None] * 10
    s[0] = g0
    for b in range(1, 9):
        ang = b * math.pi / _ORI
        s[b] = jnp.float32(math.cos(ang)) * g0 - jnp.float32(math.sin(ang)) * g1
    s[9] = -g0

    ge = [v >= 0 for v in s]
    f = [ge[b] != ge[b + 1] for b in range(_ORI)]

    bnd = g0 == 0.0

    mh = mag.astype(jnp.bfloat16).astype(jnp.float32)
    ml = mag - mh
    zero = jnp.zeros_like(mag)

    masks = [f[0] | bnd] + [f[b] for b in range(1, 8)] + [f[8] & (~bnd), bnd]

    def rowpool(m):
        uh = jnp.where(m, mh, zero)
        ul = jnp.where(m, ml, zero)
        return (jnp.dot(PT, uh, preferred_element_type=jnp.float32)
                + jnp.dot(PT, ul, preferred_element_type=jnp.float32))

    pools = [rowpool(m) for m in masks]
    R = pools[:_ORI]
    Rz = pools[_ORI]

    inv = jnp.float32(1.0 / (_CH * _CH))
    for b in range(_ORI):
        if b == 0:
            t = R[0] + R[8] + Rz
        elif b == 1:
            t = R[1] + R[0] - Rz
        else:
            t = R[b] + R[b - 1]
        th = t.astype(jnp.bfloat16).astype(jnp.float32)
        tl = t - th
        o_ref[0, b] = (
            jnp.dot(th, P, preferred_element_type=jnp.float32)
            + jnp.dot(tl, P, preferred_element_type=jnp.float32)) * inv


def kernel(x, weight):
    n = x.shape[0]
    xb = x
    i = jnp.arange(_H, dtype=jnp.int32)
    d = i[:, None] - i[None, :]
    one = jnp.float32(1.0)
    zero = jnp.float32(0.0)
    T = (jnp.where(jnp.abs(d) == 1, one, zero)
         + jnp.where(d == 0, jnp.float32(2.0), zero))
    Dr = jnp.where(d == 1, one, zero) - jnp.where(d == -1, one, zero)
    pr = jnp.arange(_PH, dtype=jnp.int32)
    PT = jnp.where(i[None, :] // _CH == pr[:, None], one, zero)
    P = jnp.where(i[:, None] // _CH == pr[None, :], one, zero)

    pooled = pl.pallas_call(
        _hog_body,
        grid=(n,),
        in_specs=[
            pl.BlockSpec((1, 1, _H, _W), lambda i: (i, 0, 0, 0)),
            pl.BlockSpec((_H, _H), lambda i: (0, 0)),
            pl.BlockSpec((_H, _H), lambda i: (0, 0)),
            pl.BlockSpec((_PH, _H), lambda i: (0, 0)),
            pl.BlockSpec((_H, _PW), lambda i: (0, 0)),
        ],
        out_specs=pl.BlockSpec((1, _ORI, _PH, _PW), lambda i: (i, 0, 0, 0)),
        out_shape=jax.ShapeDtypeStruct((n, _ORI, _PH, _PW), jnp.float32),
    )(xb, T, Dr, PT, P)
    return pooled.reshape(n, -1)
